# baseline (speedup 1.0000x reference)
import functools

import jax
import jax.numpy as jnp
from jax.experimental import pallas as pl
from jax.experimental.pallas import tpu as pltpu


def _pick_block(e):
    for cand in (1024, 1000, 800, 640, 512, 400, 320, 256, 200, 160, 128, 64, 32, 16, 8):
        if e % cand == 0:
            return cand
    return e


def _lstm_body(ssrc_ref, sdst_ref, attr_ref, wgcn_ref, bgcn_ref, wsrc_ref,
               wdst_ref, wattr_ref, bias_ref, whh_ref, wlin_ref, blin_ref,
               out_ref, h_scr, c_scr, hs_scr, pre_scr, *, be, lh):
    pi = pl.program_id(0)

    @pl.when(pi == 0)
    def _init():
        h_scr[...] = jnp.zeros_like(h_scr)
        c_scr[...] = jnp.zeros_like(c_scr)

    wgcn = wgcn_ref[...]
    bgcn = bgcn_ref[...]
    fs = jnp.maximum(ssrc_ref[...] * wgcn + bgcn, 0.0)
    fd = jnp.maximum(sdst_ref[...] * wgcn + bgcn, 0.0)
    pre = (jnp.dot(fs, wsrc_ref[...], preferred_element_type=jnp.float32)
           + jnp.dot(fd, wdst_ref[...], preferred_element_type=jnp.float32)
           + jnp.dot(attr_ref[...], wattr_ref[...],
                     preferred_element_type=jnp.float32)
           + bias_ref[...])
    pre_scr[...] = pre

    whh = whh_ref[...].astype(jnp.bfloat16)
    lane = jax.lax.broadcasted_iota(jnp.int32, (1, 4 * lh), 1)
    is_g = (lane >= 2 * lh) & (lane < 3 * lh)
    sv = jnp.where(is_g, 1.0, 0.5)
    pa = sv
    pb = jnp.where(is_g, 0.0, 0.5)

    def chunk(k, carry):
        h, c = carry
        rows = pre_scr[pl.ds(k * 8, 8), :]
        for j in range(8):
            hb = h.astype(jnp.bfloat16)
            hl = (h - hb.astype(jnp.float32)).astype(jnp.bfloat16)
            gates = (rows[j:j + 1, :]
                     + jnp.dot(hb, whh, preferred_element_type=jnp.float32)
                     + jnp.dot(hl, whh, preferred_element_type=jnp.float32))
            act = jnp.tanh(gates * sv) * pa + pb
            ii = act[:, 0:lh]
            ff = act[:, lh:2 * lh]
            gg = act[:, 2 * lh:3 * lh]
            oo = act[:, 3 * lh:4 * lh]
            c = ff * c + ii * gg
            h = oo * jnp.tanh(c)
            hs_scr[pl.ds(k * 8 + j, 1), :] = h
        return (h, c)

    hN, cN = jax.lax.fori_loop(0, be // 8, chunk, (h_scr[...], c_scr[...]))
    h_scr[...] = hN
    c_scr[...] = cN
    out_ref[...] = jnp.dot(hs_scr[...], wlin_ref[...],
                           preferred_element_type=jnp.float32) + blin_ref[...]


def _edge_lstm(s_src, s_dst, attr_p, W_gcn, b_gcn, Wsrc_T, Wdst_T, Wattr_T,
               bias, Whh_T, Wlin_T, blin):
    e = s_src.shape[0]
    h = W_gcn.shape[1]
    lh = Whh_T.shape[0]
    g4 = 4 * lh
    ap = attr_p.shape[1]
    be = _pick_block(e)
    nb = e // be

    body = functools.partial(_lstm_body, be=be, lh=lh)
    out = pl.pallas_call(
        body,
        grid=(nb,),
        in_specs=[
            pl.BlockSpec((be, 1), lambda i: (i, 0)),
            pl.BlockSpec((be, 1), lambda i: (i, 0)),
            pl.BlockSpec((be, ap), lambda i: (i, 0)),
            pl.BlockSpec((1, h), lambda i: (0, 0)),
            pl.BlockSpec((1, h), lambda i: (0, 0)),
            pl.BlockSpec((h, g4), lambda i: (0, 0)),
            pl.BlockSpec((h, g4), lambda i: (0, 0)),
            pl.BlockSpec((ap, g4), lambda i: (0, 0)),
            pl.BlockSpec((1, g4), lambda i: (0, 0)),
            pl.BlockSpec((lh, g4), lambda i: (0, 0)),
            pl.BlockSpec((lh, 1), lambda i: (0, 0)),
            pl.BlockSpec((1, 1), lambda i: (0, 0)),
        ],
        out_specs=pl.BlockSpec((be, 1), lambda i: (i, 0)),
        out_shape=jax.ShapeDtypeStruct((e, 1), jnp.float32),
        scratch_shapes=[
            pltpu.VMEM((1, lh), jnp.float32),
            pltpu.VMEM((1, lh), jnp.float32),
            pltpu.VMEM((be, lh), jnp.float32),
            pltpu.VMEM((be, g4), jnp.float32),
        ],
    )(s_src, s_dst, attr_p, W_gcn, b_gcn, Wsrc_T, Wdst_T, Wattr_T, bias,
      Whh_T, Wlin_T, blin)
    return out


def kernel(x, edge_index, edge_attr, W_gcn, b_gcn, W_ih, W_hh, b_ih, b_hh,
           W_lin, b_lin):
    n = x.shape[0]
    e = edge_index.shape[1]
    h = W_gcn.shape[1]
    lh = W_hh.shape[1]
    a = edge_attr.shape[1]

    src = edge_index[0]
    dst = edge_index[1]

    xs = x[:, 0]
    deg = jnp.zeros((n,), jnp.float32).at[dst].add(1.0) + 1.0
    dinv = jax.lax.rsqrt(deg)
    av = xs * dinv
    spart = jnp.zeros((n,), jnp.float32).at[dst].add(av[src])
    s = dinv * spart + xs * dinv * dinv
    s_src = s[src][:, None]
    s_dst = s[dst][:, None]

    attr_p = jnp.pad(edge_attr, ((0, 0), (0, 8 - a)))
    Wsrc_T = W_ih[:, :h].T
    Wdst_T = W_ih[:, h:2 * h].T
    Wattr_T = jnp.pad(W_ih[:, 2 * h:].T, ((0, 8 - a), (0, 0)))
    bias = (b_ih + b_hh)[None, :]
    Whh_T = W_hh.T
    Wlin_T = W_lin.T
    blin = b_lin[None, :]

    out = _edge_lstm(s_src, s_dst, attr_p, W_gcn, b_gcn[None, :], Wsrc_T,
                     Wdst_T, Wattr_T, bias, Whh_T, Wlin_T, blin)
    return out.reshape(-1)


# speedup vs baseline: 1.0211x; 1.0211x over previous
import functools

import jax
import jax.numpy as jnp
from jax.experimental import pallas as pl
from jax.experimental.pallas import tpu as pltpu


def _pick_block(e):
    for cand in (1024, 1000, 800, 640, 512, 400, 320, 256, 200, 160, 128, 64, 32, 16, 8):
        if e % cand == 0:
            return cand
    return e


def _lstm_body(ssrc_ref, sdst_ref, attr_ref, wgcn_ref, bgcn_ref, wsrc_ref,
               wdst_ref, wattr_ref, bias_ref, whh_ref, wlin_ref, blin_ref,
               out_ref, h_scr, c_scr, hs_scr, pre_scr, *, be, lh):
    pi = pl.program_id(0)

    @pl.when(pi == 0)
    def _init():
        h_scr[...] = jnp.zeros_like(h_scr)
        c_scr[...] = jnp.zeros_like(c_scr)

    wgcn = wgcn_ref[...]
    bgcn = bgcn_ref[...]
    fs = jnp.maximum(ssrc_ref[...] * wgcn + bgcn, 0.0)
    fd = jnp.maximum(sdst_ref[...] * wgcn + bgcn, 0.0)
    pre = (jnp.dot(fs, wsrc_ref[...], preferred_element_type=jnp.float32)
           + jnp.dot(fd, wdst_ref[...], preferred_element_type=jnp.float32)
           + jnp.dot(attr_ref[...], wattr_ref[...],
                     preferred_element_type=jnp.float32)
           + bias_ref[...])
    pre_scr[...] = pre

    whh = whh_ref[...]
    lane = jax.lax.broadcasted_iota(jnp.int32, (1, 4 * lh), 1)
    is_g = (lane >= 2 * lh) & (lane < 3 * lh)
    sv = jnp.where(is_g, 1.0, 0.5)
    pa = sv
    pb = jnp.where(is_g, 0.0, 0.5)

    def chunk(k, carry):
        h8, c = carry
        rows = pre_scr[pl.ds(k * 8, 8), :]
        for j in range(8):
            mm = jnp.dot(h8, whh,
                         preferred_element_type=jnp.float32)
            gates = rows[j:j + 1, :] + mm[0:1, :]
            act = jnp.tanh(gates * sv) * pa + pb
            ii = act[:, 0:lh]
            ff = act[:, lh:2 * lh]
            gg = act[:, 2 * lh:3 * lh]
            oo = act[:, 3 * lh:4 * lh]
            c = ff * c + ii * gg
            hrow = oo * jnp.tanh(c)
            hs_scr[pl.ds(k * 8 + j, 1), :] = hrow
            h8 = jnp.broadcast_to(hrow, (8, lh))
        return (h8, c)

    h80 = jnp.broadcast_to(h_scr[...], (8, lh))
    h8N, cN = jax.lax.fori_loop(0, be // 8, chunk, (h80, c_scr[...]))
    h_scr[...] = h8N[0:1, :]
    c_scr[...] = cN
    out_ref[...] = jnp.dot(hs_scr[...], wlin_ref[...],
                           preferred_element_type=jnp.float32) + blin_ref[...]


def _edge_lstm(s_src, s_dst, attr_p, W_gcn, b_gcn, Wsrc_T, Wdst_T, Wattr_T,
               bias, Whh_T, Wlin_T, blin):
    e = s_src.shape[0]
    h = W_gcn.shape[1]
    lh = Whh_T.shape[0]
    g4 = 4 * lh
    ap = attr_p.shape[1]
    be = _pick_block(e)
    nb = e // be

    body = functools.partial(_lstm_body, be=be, lh=lh)
    out = pl.pallas_call(
        body,
        grid=(nb,),
        in_specs=[
            pl.BlockSpec((be, 1), lambda i: (i, 0)),
            pl.BlockSpec((be, 1), lambda i: (i, 0)),
            pl.BlockSpec((be, ap), lambda i: (i, 0)),
            pl.BlockSpec((1, h), lambda i: (0, 0)),
            pl.BlockSpec((1, h), lambda i: (0, 0)),
            pl.BlockSpec((h, g4), lambda i: (0, 0)),
            pl.BlockSpec((h, g4), lambda i: (0, 0)),
            pl.BlockSpec((ap, g4), lambda i: (0, 0)),
            pl.BlockSpec((1, g4), lambda i: (0, 0)),
            pl.BlockSpec((lh, g4), lambda i: (0, 0)),
            pl.BlockSpec((lh, 1), lambda i: (0, 0)),
            pl.BlockSpec((1, 1), lambda i: (0, 0)),
        ],
        out_specs=pl.BlockSpec((be, 1), lambda i: (i, 0)),
        out_shape=jax.ShapeDtypeStruct((e, 1), jnp.float32),
        scratch_shapes=[
            pltpu.VMEM((1, lh), jnp.float32),
            pltpu.VMEM((1, lh), jnp.float32),
            pltpu.VMEM((be, lh), jnp.float32),
            pltpu.VMEM((be, g4), jnp.float32),
        ],
    )(s_src, s_dst, attr_p, W_gcn, b_gcn, Wsrc_T, Wdst_T, Wattr_T, bias,
      Whh_T, Wlin_T, blin)
    return out


def kernel(x, edge_index, edge_attr, W_gcn, b_gcn, W_ih, W_hh, b_ih, b_hh,
           W_lin, b_lin):
    n = x.shape[0]
    e = edge_index.shape[1]
    h = W_gcn.shape[1]
    lh = W_hh.shape[1]
    a = edge_attr.shape[1]

    src = edge_index[0]
    dst = edge_index[1]

    xs = x[:, 0]
    deg = jnp.zeros((n,), jnp.float32).at[dst].add(1.0) + 1.0
    dinv = jax.lax.rsqrt(deg)
    av = xs * dinv
    spart = jnp.zeros((n,), jnp.float32).at[dst].add(av[src])
    s = dinv * spart + xs * dinv * dinv
    s_src = s[src][:, None]
    s_dst = s[dst][:, None]

    attr_p = jnp.pad(edge_attr, ((0, 0), (0, 8 - a)))
    Wsrc_T = W_ih[:, :h].T
    Wdst_T = W_ih[:, h:2 * h].T
    Wattr_T = jnp.pad(W_ih[:, 2 * h:].T, ((0, 8 - a), (0, 0)))
    bias = (b_ih + b_hh)[None, :]
    Whh_T = W_hh.T
    Wlin_T = W_lin.T
    blin = b_lin[None, :]

    out = _edge_lstm(s_src, s_dst, attr_p, W_gcn, b_gcn[None, :], Wsrc_T,
                     Wdst_T, Wattr_T, bias, Whh_T, Wlin_T, blin)
    return out.reshape(-1)


# speedup vs baseline: 1.7385x; 1.7027x over previous
import functools

import jax
import jax.numpy as jnp
from jax.experimental import pallas as pl
from jax.experimental.pallas import tpu as pltpu


def _pick_block(e):
    for cand in (1024, 1000, 800, 640, 512, 400, 320, 256, 200, 160, 128, 64, 32, 16, 8):
        if e % cand == 0:
            return cand
    return e


def _lstm_body(ssrc_ref, sdst_ref, attr_ref, wgcn_ref, bgcn_ref, wsrc_ref,
               wdst_ref, wattr_ref, bias_ref, whh_ref, wlin_ref, blin_ref,
               out_ref, h_scr, c_scr, hs_scr, pi_scr, pf_scr, pg_scr, po_scr,
               *, be, lh):
    pid = pl.program_id(0)

    @pl.when(pid == 0)
    def _init():
        h_scr[...] = jnp.zeros_like(h_scr)
        c_scr[...] = jnp.zeros_like(c_scr)

    wgcn = wgcn_ref[...]
    bgcn = bgcn_ref[...]
    fs = jnp.maximum(ssrc_ref[...] * wgcn + bgcn, 0.0)
    fd = jnp.maximum(sdst_ref[...] * wgcn + bgcn, 0.0)
    attr = attr_ref[...]

    whh = whh_ref[...]
    whh_g = []
    gate_scrs = (pi_scr, pf_scr, pg_scr, po_scr)
    for g in range(4):
        sl = slice(g * lh, (g + 1) * lh)
        gate_scrs[g][...] = (
            jnp.dot(fs, wsrc_ref[:, sl], preferred_element_type=jnp.float32)
            + jnp.dot(fd, wdst_ref[:, sl], preferred_element_type=jnp.float32)
            + jnp.dot(attr, wattr_ref[:, sl],
                      preferred_element_type=jnp.float32)
            + bias_ref[:, sl])
        whh_g.append(whh[:, sl])

    def chunk(k, carry):
        h8, c = carry
        ri = pi_scr[pl.ds(k * 8, 8), :]
        rf = pf_scr[pl.ds(k * 8, 8), :]
        rg = pg_scr[pl.ds(k * 8, 8), :]
        ro = po_scr[pl.ds(k * 8, 8), :]
        for j in range(8):
            mi = jnp.dot(h8, whh_g[0], preferred_element_type=jnp.float32)
            mf = jnp.dot(h8, whh_g[1], preferred_element_type=jnp.float32)
            mg = jnp.dot(h8, whh_g[2], preferred_element_type=jnp.float32)
            mo = jnp.dot(h8, whh_g[3], preferred_element_type=jnp.float32)
            iact = 0.5 * jnp.tanh(0.5 * (ri[j:j + 1, :] + mi[0:1, :])) + 0.5
            fact = 0.5 * jnp.tanh(0.5 * (rf[j:j + 1, :] + mf[0:1, :])) + 0.5
            gact = jnp.tanh(rg[j:j + 1, :] + mg[0:1, :])
            oact = 0.5 * jnp.tanh(0.5 * (ro[j:j + 1, :] + mo[0:1, :])) + 0.5
            c = fact * c + iact * gact
            hrow = oact * jnp.tanh(c)
            hs_scr[pl.ds(k * 8 + j, 1), :] = hrow
            h8 = jnp.broadcast_to(hrow, (8, lh))
        return (h8, c)

    h80 = jnp.broadcast_to(h_scr[...], (8, lh))
    h8N, cN = jax.lax.fori_loop(0, be // 8, chunk, (h80, c_scr[...]))
    h_scr[...] = h8N[0:1, :]
    c_scr[...] = cN
    out_ref[...] = jnp.dot(hs_scr[...], wlin_ref[...],
                           preferred_element_type=jnp.float32) + blin_ref[...]


def _edge_lstm(s_src, s_dst, attr_p, W_gcn, b_gcn, Wsrc_T, Wdst_T, Wattr_T,
               bias, Whh_T, Wlin_T, blin):
    e = s_src.shape[0]
    h = W_gcn.shape[1]
    lh = Whh_T.shape[0]
    g4 = 4 * lh
    ap = attr_p.shape[1]
    be = _pick_block(e)
    nb = e // be

    body = functools.partial(_lstm_body, be=be, lh=lh)
    out = pl.pallas_call(
        body,
        grid=(nb,),
        in_specs=[
            pl.BlockSpec((be, 1), lambda i: (i, 0)),
            pl.BlockSpec((be, 1), lambda i: (i, 0)),
            pl.BlockSpec((be, ap), lambda i: (i, 0)),
            pl.BlockSpec((1, h), lambda i: (0, 0)),
            pl.BlockSpec((1, h), lambda i: (0, 0)),
            pl.BlockSpec((h, g4), lambda i: (0, 0)),
            pl.BlockSpec((h, g4), lambda i: (0, 0)),
            pl.BlockSpec((ap, g4), lambda i: (0, 0)),
            pl.BlockSpec((1, g4), lambda i: (0, 0)),
            pl.BlockSpec((lh, g4), lambda i: (0, 0)),
            pl.BlockSpec((lh, 1), lambda i: (0, 0)),
            pl.BlockSpec((1, 1), lambda i: (0, 0)),
        ],
        out_specs=pl.BlockSpec((be, 1), lambda i: (i, 0)),
        out_shape=jax.ShapeDtypeStruct((e, 1), jnp.float32),
        scratch_shapes=[
            pltpu.VMEM((1, lh), jnp.float32),
            pltpu.VMEM((1, lh), jnp.float32),
            pltpu.VMEM((be, lh), jnp.float32),
            pltpu.VMEM((be, lh), jnp.float32),
            pltpu.VMEM((be, lh), jnp.float32),
            pltpu.VMEM((be, lh), jnp.float32),
            pltpu.VMEM((be, lh), jnp.float32),
        ],
    )(s_src, s_dst, attr_p, W_gcn, b_gcn, Wsrc_T, Wdst_T, Wattr_T, bias,
      Whh_T, Wlin_T, blin)
    return out


def kernel(x, edge_index, edge_attr, W_gcn, b_gcn, W_ih, W_hh, b_ih, b_hh,
           W_lin, b_lin):
    n = x.shape[0]
    e = edge_index.shape[1]
    h = W_gcn.shape[1]
    lh = W_hh.shape[1]
    a = edge_attr.shape[1]

    src = edge_index[0]
    dst = edge_index[1]

    xs = x[:, 0]
    deg = jnp.zeros((n,), jnp.float32).at[dst].add(1.0) + 1.0
    dinv = jax.lax.rsqrt(deg)
    av = xs * dinv
    spart = jnp.zeros((n,), jnp.float32).at[dst].add(av[src])
    s = dinv * spart + xs * dinv * dinv
    s_src = s[src][:, None]
    s_dst = s[dst][:, None]

    attr_p = jnp.pad(edge_attr, ((0, 0), (0, 8 - a)))
    Wsrc_T = W_ih[:, :h].T
    Wdst_T = W_ih[:, h:2 * h].T
    Wattr_T = jnp.pad(W_ih[:, 2 * h:].T, ((0, 8 - a), (0, 0)))
    bias = (b_ih + b_hh)[None, :]
    Whh_T = W_hh.T
    Wlin_T = W_lin.T
    blin = b_lin[None, :]

    out = _edge_lstm(s_src, s_dst, attr_p, W_gcn, b_gcn[None, :], Wsrc_T,
                     Wdst_T, Wattr_T, bias, Whh_T, Wlin_T, blin)
    return out.reshape(-1)
